# baseline (speedup 1.0000x reference)
import jax
import jax.numpy as jnp
from jax import lax
from jax.experimental import pallas as pl
from jax.experimental.pallas import tpu as pltpu
from jax.experimental.pallas import tpu_sc as plsc

B = 8
N = 2048
MC = 1024
NMC = N // MC
BIG = 2**30

NC = 2
NS = 16
NW = NC * NS
ROWS_PER_W = N // NW
GROUPS = ROWS_PER_W // 16


def _tc_score_body(x_ref, y_ref, o1_ref, o2_ref):
    mc = pl.program_id(1)
    xb = x_ref[0]
    yb = y_ref[0]
    xc = x_ref[0, pl.ds(mc * MC, MC), :]
    yc = y_ref[0, pl.ds(mc * MC, MC), :]
    nxb = jnp.sum(xb * xb, axis=1, keepdims=True)
    nyb = jnp.sum(yb * yb, axis=1, keepdims=True)
    ones_c = jnp.ones((MC, 1), jnp.float32)

    ya = jnp.concatenate([yb, -0.5 * nyb], axis=1)
    xc1 = jnp.concatenate([xc, ones_c], axis=1)
    sc1 = lax.dot_general(ya, xc1, (((1,), (1,)), ((), ())),
                          preferred_element_type=jnp.float32)
    o1_ref[...] = jnp.argmax(sc1, axis=0).astype(jnp.int32).reshape(1, 1, 1, MC)

    xa = jnp.concatenate([xb, -0.5 * nxb], axis=1)
    yc1 = jnp.concatenate([yc, ones_c], axis=1)
    sc2 = lax.dot_general(xa, yc1, (((1,), (1,)), ((), ())),
                          preferred_element_type=jnp.float32)
    o2_ref[...] = jnp.argmax(sc2, axis=0).astype(jnp.int32).reshape(1, 1, 1, MC)


def _sc_gather_body(xyf_hbm, d1_hbm, d2_hbm, out_hbm,
                    xys, i1, i2, res, sem):
    wid = lax.axis_index("s") * NC + lax.axis_index("c")
    row_base = wid * ROWS_PER_W

    mcw = row_base // MC
    off = row_base % MC
    cp = [
        pltpu.async_copy(xyf_hbm, xys, sem),
    ]
    for b in range(B):
        src = (b, mcw, 0, pl.ds(off, ROWS_PER_W))
        dst = pl.ds(b * ROWS_PER_W, ROWS_PER_W)
        cp.append(pltpu.async_copy(d1_hbm.at[src], i1.at[dst], sem))
        cp.append(pltpu.async_copy(d2_hbm.at[src], i2.at[dst], sem))
    for c in cp:
        c.wait()

    lane3 = lax.iota(jnp.int32, 16) * 3

    def one_direction(rbase, pbase, idx, b, ctx):
        total = jnp.zeros((16,), jnp.float32)
        for g in range(GROUPS):
            iv = idx[pl.ds(b * ROWS_PER_W + g * 16, 16)]
            wflat = pbase + iv * 3
            rflat = rbase + (row_base + g * 16) * 3 + lane3
            f5 = jnp.zeros((16,), jnp.float32)
            for d in range(3):
                w = plsc.load_gather(xys, [wflat + d])
                r = plsc.load_gather(xys, [rflat + d])
                a = jnp.abs(r - w)
                a2 = a * a
                f5 = f5 + a2 * a2 * a
            total = total + f5
        res[ctx, :] = total

    def per_batch(b, carry):
        xb = b * (3 * N)
        yb = B * N * 3 + b * (3 * N)
        one_direction(xb, yb, i1, b, b)
        one_direction(yb, xb, i2, b, b + B)
        return carry

    lax.fori_loop(0, B, per_batch, 0)
    pltpu.sync_copy(res, out_hbm.at[wid])


def _epilogue_body(parts_ref, out_ref):
    s = jnp.sum(parts_ref[...], axis=1, keepdims=True)
    out_ref[...] = jnp.sum(s ** 0.2, axis=(0, 1), keepdims=True) * (1.0 / B)


@jax.jit
def kernel(x, y):
    d1, d2 = pl.pallas_call(
        _tc_score_body,
        grid=(B, NMC),
        in_specs=[
            pl.BlockSpec((1, N, 3), lambda b, mc: (b, 0, 0)),
            pl.BlockSpec((1, N, 3), lambda b, mc: (b, 0, 0)),
        ],
        out_specs=[
            pl.BlockSpec((1, 1, 1, MC), lambda b, mc: (b, mc, 0, 0)),
            pl.BlockSpec((1, 1, 1, MC), lambda b, mc: (b, mc, 0, 0)),
        ],
        out_shape=[
            jax.ShapeDtypeStruct((B, NMC, 1, MC), jnp.int32),
            jax.ShapeDtypeStruct((B, NMC, 1, MC), jnp.int32),
        ],
        compiler_params=pltpu.CompilerParams(
            dimension_semantics=("arbitrary", "arbitrary")),
    )(x, y)
    xyf = jnp.concatenate([x.reshape(B * N * 3), y.reshape(B * N * 3)])

    mesh = plsc.VectorSubcoreMesh(core_axis_name="c", subcore_axis_name="s",
                                  num_cores=NC, num_subcores=NS)
    sc_call = pl.kernel(
        _sc_gather_body,
        out_type=jax.ShapeDtypeStruct((NW, 2 * B, 16), jnp.float32),
        mesh=mesh,
        compiler_params=pltpu.CompilerParams(needs_layout_passes=False),
        scratch_types=[
            pltpu.VMEM((2 * B * N * 3,), jnp.float32),
            pltpu.VMEM((B * ROWS_PER_W,), jnp.int32),
            pltpu.VMEM((B * ROWS_PER_W,), jnp.int32),
            pltpu.VMEM((2 * B, 16), jnp.float32),
            pltpu.SemaphoreType.DMA,
        ],
    )
    parts = sc_call(xyf, d1, d2)

    parts2 = jnp.transpose(parts, (1, 0, 2)).reshape(2 * B, NW * 16)
    out = pl.pallas_call(
        _epilogue_body,
        out_shape=jax.ShapeDtypeStruct((1, 1), jnp.float32),
    )(parts2)
    return out[0, 0]
